# baseline (speedup 1.0000x reference)
import functools

import jax
import jax.numpy as jnp
from jax import lax
from jax.experimental import pallas as pl
from jax.experimental.pallas import tpu as pltpu
from jax.experimental.pallas import tpu_sc as plsc

D = 1024
B = 4096
NROW = 71
NC, NS = 2, 16
NW = NC * NS
RPW = (B * NROW) // NW

NBLK = 38
GROWS = NBLK * 8 * 64
SMALL0 = 37 * 8 * 64

CH = 16
NB = 4
LAG = 1
NCHUNK = RPW // CH
NGRP = NCHUNK // NB


def _build_body(pair_ref, add_ref, out_ref):
    out_ref[...] = pair_ref[...] + add_ref[...]


def _build_table(pair_ext, addend3):
    out3 = pl.pallas_call(
        _build_body,
        grid=(NBLK,),
        in_specs=[
            pl.BlockSpec((8, 1, D), lambda k: (k, 0, 0)),
            pl.BlockSpec((8, 64, D), lambda k: (jnp.where(k < 37, 0, 1), 0, 0)),
        ],
        out_specs=pl.BlockSpec((8, 64, D), lambda k: (k, 0, 0)),
        out_shape=jax.ShapeDtypeStruct((NBLK * 8, 64, D), jnp.float32),
    )(pair_ext.reshape(NBLK * 8, 1, D), addend3)
    return out3.reshape(GROWS, D)


def _sc_body(G2, idxs, out, idx_v, bufs, gsem, wsem):
    wid = lax.axis_index("s") * NC + lax.axis_index("c")
    row0 = wid * RPW
    pltpu.sync_copy(idxs.at[pl.ds(row0, RPW)], idx_v)

    def gather(i, s):
        off = pl.multiple_of(i * CH, CH)
        return pltpu.make_async_copy(
            G2.at[idx_v.at[pl.ds(off, CH)]], bufs.at[s], gsem.at[s])

    def write(i, s):
        off = pl.multiple_of(row0 + i * CH, CH)
        return pltpu.make_async_copy(
            bufs.at[s], out.at[pl.ds(off, CH)], wsem.at[s])

    for s in range(LAG):
        gather(s, s).start()

    def group(g, _):
        i0 = g * NB
        for s in range(NB):
            i = i0 + s
            s2 = (s + LAG) % NB

            @pl.when(i + LAG < NCHUNK)
            def _():
                @pl.when(i + LAG >= NB)
                def _():
                    write(i + LAG - NB, s2).wait()
                gather(i + LAG, s2).start()

            gather(i, s).wait()
            write(i, s).start()
        return 0

    lax.fori_loop(0, NGRP, group, 0)
    for s in range(NB):
        write(NCHUNK - NB + s, s).wait()


@functools.partial(
    pl.kernel,
    out_type=jax.ShapeDtypeStruct((NROW * B, D), jnp.float32),
    mesh=plsc.VectorSubcoreMesh(
        core_axis_name="c", subcore_axis_name="s", num_cores=NC, num_subcores=NS
    ),
    compiler_params=pltpu.CompilerParams(use_tc_tiling_on_sc=True),
    scratch_types=[
        pltpu.VMEM((RPW,), jnp.int32),
        pltpu.VMEM((NB, CH, D), jnp.float32),
        pltpu.SemaphoreType.DMA((NB,)),
        pltpu.SemaphoreType.DMA((NB,)),
    ],
)
def _sc_embed(G2, idxs, out, idx_v, bufs, gsem, wsem):
    _sc_body(G2, idxs, out, idx_v, bufs, gsem, wsem)


def kernel(fen, move, rank_emb, file_emb, fen_emb, move_emb, abs_emb):
    pos = (rank_emb + file_emb).reshape(64, D)
    pair = 0.5 * (fen_emb[:, None, :] + fen_emb[None, :, :]).reshape(289, D)
    boardc = 0.5 * pos + abs_emb[:64]
    flag_tab = (fen_emb[None, :, :] + abs_emb[64:69][:, None, :]).reshape(85, D)
    mv_tab = (
        0.58 * (pos[None, :, :] + move_emb[:, None, :])
        + abs_emb[69:71][:, None, :]
    ).reshape(128, D)
    pair_ext = jnp.concatenate(
        [pair, jnp.zeros((NBLK * 8 - 289, D), jnp.float32)])
    small_pad = jnp.concatenate(
        [flag_tab, mv_tab, jnp.zeros((512 - 85 - 128, D), jnp.float32)])
    addend3 = jnp.stack(
        [jnp.broadcast_to(boardc, (8, 64, D)), small_pad.reshape(8, 64, D)]
    ).reshape(16, 64, D)
    G2 = _build_table(pair_ext, addend3)

    iota64 = jnp.arange(64, dtype=jnp.int32)
    idx_board = (fen[:, :64] * 17 + fen[:, 64:128]) * 64 + iota64
    idx_flag = SMALL0 + jnp.arange(5, dtype=jnp.int32) * 17 + fen[:, 128:133]
    idx_mv = SMALL0 + 85 + jnp.arange(2, dtype=jnp.int32) * 64 + move
    idx = jnp.concatenate(
        [idx_board, idx_flag, idx_mv], axis=1).T.reshape(-1)

    out = _sc_embed(G2, idx)
    return out.reshape(NROW, B, D).transpose(1, 0, 2)


# speedup vs baseline: 1.0107x; 1.0107x over previous
import functools

import jax
import jax.numpy as jnp
from jax import lax
from jax.experimental import pallas as pl
from jax.experimental.pallas import tpu as pltpu
from jax.experimental.pallas import tpu_sc as plsc

D = 1024
B = 4096
NROW = 71
NC, NS = 2, 16
NW = NC * NS
RPW = (B * NROW) // NW

NBLK = 38
GROWS = NBLK * 8 * 64
SMALL0 = 37 * 8 * 64

CH = 16
NB = 4
LAG = 2
NCHUNK = RPW // CH
NGRP = NCHUNK // NB


def _build_body(pair_ref, add_ref, out_ref):
    out_ref[...] = pair_ref[...] + add_ref[...]


def _build_table(pair_ext, addend3):
    out3 = pl.pallas_call(
        _build_body,
        grid=(NBLK,),
        in_specs=[
            pl.BlockSpec((8, 1, D), lambda k: (k, 0, 0)),
            pl.BlockSpec((8, 64, D), lambda k: (jnp.where(k < 37, 0, 1), 0, 0)),
        ],
        out_specs=pl.BlockSpec((8, 64, D), lambda k: (k, 0, 0)),
        out_shape=jax.ShapeDtypeStruct((NBLK * 8, 64, D), jnp.float32),
    )(pair_ext.reshape(NBLK * 8, 1, D), addend3)
    return out3.reshape(GROWS, D)


def _sc_body(G2, idxs, out, idx_v, bufs, gsem, wsem):
    wid = lax.axis_index("s") * NC + lax.axis_index("c")
    row0 = wid * RPW
    pltpu.sync_copy(idxs.at[pl.ds(row0, RPW)], idx_v)

    def gather(i, s):
        off = pl.multiple_of(i * CH, CH)
        return pltpu.make_async_copy(
            G2.at[idx_v.at[pl.ds(off, CH)]], bufs.at[s], gsem.at[s])

    def write(i, s):
        off = pl.multiple_of(row0 + i * CH, CH)
        return pltpu.make_async_copy(
            bufs.at[s], out.at[pl.ds(off, CH)], wsem.at[s])

    for s in range(LAG):
        gather(s, s).start()

    def group(g, _):
        i0 = g * NB
        for s in range(NB):
            i = i0 + s
            s2 = (s + LAG) % NB

            @pl.when(i + LAG < NCHUNK)
            def _():
                @pl.when(i + LAG >= NB)
                def _():
                    write(i + LAG - NB, s2).wait()
                gather(i + LAG, s2).start()

            gather(i, s).wait()
            write(i, s).start()
        return 0

    lax.fori_loop(0, NGRP, group, 0)
    for s in range(NB):
        write(NCHUNK - NB + s, s).wait()


@functools.partial(
    pl.kernel,
    out_type=jax.ShapeDtypeStruct((NROW * B, D), jnp.float32),
    mesh=plsc.VectorSubcoreMesh(
        core_axis_name="c", subcore_axis_name="s", num_cores=NC, num_subcores=NS
    ),
    compiler_params=pltpu.CompilerParams(use_tc_tiling_on_sc=True),
    scratch_types=[
        pltpu.VMEM((RPW,), jnp.int32),
        pltpu.VMEM((NB, CH, D), jnp.float32),
        pltpu.SemaphoreType.DMA((NB,)),
        pltpu.SemaphoreType.DMA((NB,)),
    ],
)
def _sc_embed(G2, idxs, out, idx_v, bufs, gsem, wsem):
    _sc_body(G2, idxs, out, idx_v, bufs, gsem, wsem)


def kernel(fen, move, rank_emb, file_emb, fen_emb, move_emb, abs_emb):
    pos = (rank_emb + file_emb).reshape(64, D)
    pair = 0.5 * (fen_emb[:, None, :] + fen_emb[None, :, :]).reshape(289, D)
    boardc = 0.5 * pos + abs_emb[:64]
    flag_tab = (fen_emb[None, :, :] + abs_emb[64:69][:, None, :]).reshape(85, D)
    mv_tab = (
        0.58 * (pos[None, :, :] + move_emb[:, None, :])
        + abs_emb[69:71][:, None, :]
    ).reshape(128, D)
    pair_ext = jnp.concatenate(
        [pair, jnp.zeros((NBLK * 8 - 289, D), jnp.float32)])
    small_pad = jnp.concatenate(
        [flag_tab, mv_tab, jnp.zeros((512 - 85 - 128, D), jnp.float32)])
    addend3 = jnp.stack(
        [jnp.broadcast_to(boardc, (8, 64, D)), small_pad.reshape(8, 64, D)]
    ).reshape(16, 64, D)
    G2 = _build_table(pair_ext, addend3)

    iota64 = jnp.arange(64, dtype=jnp.int32)
    idx_board = (fen[:, :64] * 17 + fen[:, 64:128]) * 64 + iota64
    idx_flag = SMALL0 + jnp.arange(5, dtype=jnp.int32) * 17 + fen[:, 128:133]
    idx_mv = SMALL0 + 85 + jnp.arange(2, dtype=jnp.int32) * 64 + move
    idx = jnp.concatenate(
        [idx_board, idx_flag, idx_mv], axis=1).T.reshape(-1)

    out = _sc_embed(G2, idx)
    return out.reshape(NROW, B, D).transpose(1, 0, 2)
